# Initial kernel scaffold; baseline (speedup 1.0000x reference)
import jax
import jax.numpy as jnp
from jax.experimental import pallas as pl


def kernel(x):
    raise NotImplementedError("write your pallas kernel here")


# speedup vs baseline: 26.1938x; 26.1938x over previous
import functools
import math

import jax
import jax.numpy as jnp
import numpy as np
from jax.experimental import pallas as pl

_NUM_ANGLE = 100
_NUM_RHO = 100
_A_BLK = 4


@functools.lru_cache(maxsize=None)
def _rho_table(H, W, num_angle, num_rho):
    irho = int(math.sqrt(H * H + W * W) + 1) / float(num_rho)
    itheta = math.pi / num_angle
    angles = np.arange(num_angle, dtype=np.float64) * itheta
    cosv = (np.cos(angles) / irho).astype(np.float32)
    sinv = (np.sin(angles) / irho).astype(np.float32)
    ys, xs = np.meshgrid(np.arange(H), np.arange(W), indexing="ij")
    xx = (xs - W // 2).reshape(-1).astype(np.float32)
    yy = (ys - H // 2).reshape(-1).astype(np.float32)
    r = np.round(xx[None, :] * cosv[:, None] + yy[None, :] * sinv[:, None])
    r = r.astype(np.int32) + num_rho // 2
    r = np.clip(r, 0, num_rho - 1)
    return r


def _dht_body(ridx_ref, x_ref, out_ref):
    hw = x_ref.shape[1]
    for i in range(_A_BLK):
        row = ridx_ref[0, i, :].reshape(1, hw)
        rho = jax.lax.broadcasted_iota(jnp.int32, (_NUM_RHO, hw), 0)
        onehot = (row == rho).astype(jnp.bfloat16)
        acc = jax.lax.dot_general(
            x_ref[...],
            onehot,
            dimension_numbers=(((1,), (1,)), ((), ())),
            preferred_element_type=jnp.float32,
        )
        out_ref[i, :, :] = acc


def kernel(x):
    B, C, H, W = x.shape
    BC = B * C
    HW = H * W
    ridx = jnp.asarray(
        _rho_table(H, W, _NUM_ANGLE, _NUM_RHO).reshape(
            _NUM_ANGLE // _A_BLK, _A_BLK, HW
        )
    )
    xb = x.reshape(BC, HW).astype(jnp.bfloat16)
    out = pl.pallas_call(
        _dht_body,
        grid=(_NUM_ANGLE // _A_BLK,),
        in_specs=[
            pl.BlockSpec((1, _A_BLK, HW), lambda a: (a, 0, 0)),
            pl.BlockSpec((BC, HW), lambda a: (0, 0)),
        ],
        out_specs=pl.BlockSpec((_A_BLK, BC, _NUM_RHO), lambda a: (a, 0, 0)),
        out_shape=jax.ShapeDtypeStruct((_NUM_ANGLE, BC, _NUM_RHO), jnp.float32),
    )(ridx, xb)
    return jnp.transpose(out, (1, 0, 2)).reshape(B, C, _NUM_ANGLE, _NUM_RHO)
